# Initial kernel scaffold; baseline (speedup 1.0000x reference)
import jax
import jax.numpy as jnp
from jax.experimental import pallas as pl


def kernel(inputs, categories, mask_positions, category_table, mask_category_table):
    raise NotImplementedError("write your pallas kernel here")


# speedup vs baseline: 1.4739x; 1.4739x over previous
import dataclasses
import functools

import jax
import jax.numpy as jnp
from jax import lax
from jax.experimental import pallas as pl
from jax.experimental.pallas import tpu as pltpu
from jax.experimental.pallas import tpu_sc as plsc

L = 16


def _sc_concat(in2d, cat2d, msk2d, table, mct, N, F, CDIM, NC, NS):
    NW = NC * NS
    CPW = N // NW
    CH = 256
    G = CH // 128
    NCH = CPW // CH
    assert CPW % CH == 0 and CH % 128 == 0 and CDIM % L == 0
    mesh = plsc.VectorSubcoreMesh(core_axis_name="c", subcore_axis_name="s")
    cp = pltpu.CompilerParams(use_tc_tiling_on_sc=False)
    if "needs_layout_passes" in pltpu.CompilerParams.__dataclass_fields__:
        cp = dataclasses.replace(cp, needs_layout_passes=False)

    @functools.partial(
        pl.kernel,
        out_type=jax.ShapeDtypeStruct((N, F + CDIM), jnp.float32),
        mesh=mesh,
        compiler_params=cp,
        scratch_types=[
            pltpu.VMEM((G, 128), jnp.int32),
            pltpu.VMEM((G, 128), jnp.int32),
            pltpu.VMEM((CH, CDIM), jnp.float32),
            pltpu.VMEM((CH, F), jnp.float32),
            pltpu.VMEM((1, CDIM), jnp.float32),
            pltpu.SemaphoreType.DMA,
        ],
    )
    def k(in_hbm, cat_hbm, msk_hbm, tab_hbm, mct_hbm, out_hbm,
          idx_v, msk_v, emb_v, in_v, mv_v, sem):
        wid = lax.axis_index("s") * NC + lax.axis_index("c")
        pltpu.sync_copy(mct_hbm, mv_v)
        mvs = [mv_v[0, pl.ds(c * L, L)] for c in range(CDIM // L)]

        @pl.loop(0, NCH)
        def _chunk(kk):
            base = wid * CPW + kk * CH
            rb = wid * (CPW // 128) + kk * G
            pltpu.sync_copy(cat_hbm.at[pl.ds(rb, G)], idx_v)
            pltpu.sync_copy(msk_hbm.at[pl.ds(rb, G)], msk_v)
            cps = [
                pltpu.async_copy(tab_hbm.at[idx_v.at[j]],
                                 emb_v.at[pl.ds(j * 128, 128)], sem)
                for j in range(G)
            ]
            pltpu.sync_copy(in_hbm.at[pl.ds(base, CH)], in_v)
            for cp in cps:
                cp.wait()
            for j in range(G):
                @pl.loop(0, 128 // L)
                def _grp(g8, j=j):
                    cat16 = idx_v[j, pl.ds(g8 * L, L)]
                    m16 = msk_v[j, pl.ds(g8 * L, L)]
                    keep16 = ((cat16 != 0) & (m16 != 1)).astype(jnp.float32)
                    msk16 = (m16 == 1).astype(jnp.float32)
                    row0 = j * 128 + g8 * L
                    for r in range(L):
                        kf = keep16[r]
                        bf = msk16[r]
                        for c in range(CDIM // L):
                            sl = pl.ds(c * L, L)
                            emb_v[row0 + r, sl] = (
                                emb_v[row0 + r, sl] * kf + mvs[c] * bf)
            pltpu.sync_copy(in_v, out_hbm.at[pl.ds(base, CH), pl.ds(0, F)])
            pltpu.sync_copy(emb_v, out_hbm.at[pl.ds(base, CH), pl.ds(F, CDIM)])

    return k(in2d, cat2d, msk2d, table, mct)


def kernel(inputs, categories, mask_positions, category_table,
           mask_category_table):
    B, S, F = inputs.shape
    CDIM = category_table.shape[1]
    N = B * S
    info = plsc.get_sparse_core_info()
    cat2d = categories.astype(jnp.int32).reshape(N // 128, 128)
    msk2d = mask_positions.astype(jnp.int32).reshape(N // 128, 128)
    in2d = inputs.reshape(N, F)
    out = _sc_concat(in2d, cat2d, msk2d, category_table,
                     mask_category_table, N, F, CDIM,
                     info.num_cores, info.num_subcores)
    return out.reshape(B, S, F + CDIM)
